# Initial kernel scaffold; baseline (speedup 1.0000x reference)
import functools

import jax
import jax.numpy as jnp
from jax import lax
from jax.experimental import pallas as pl
from jax.experimental.pallas import tpu as pltpu
from jax.experimental.pallas import tpu_sc as plsc

B = 16
N = 20000
FEAT = 224
W = 256
HROW = 225
ACC = HROW * W
NC = 2
NS = 16
NPAD = 20480
NWBOX = NPAD // NC
CH = 512
NCHUNK = NWBOX // CH
SUB = 32
ZB = 7200


def _sc_scatter_body(planes, out, acc, cbuf, x1b, y1b, x2b, y2b, ibuf, vbuf,
                     zbuf):
  c = lax.axis_index("c")
  s = lax.axis_index("s")
  soff = s * ACC

  def _zb(i, carry):
    zbuf[pl.ds(i * 16, 16)] = jnp.zeros((16,), jnp.float32)
    return carry

  lax.fori_loop(0, ZB // 16, _zb, 0)

  def _za(k, carry):
    pltpu.sync_copy(zbuf, acc.at[pl.ds(soff + k * ZB, ZB)])
    return carry

  lax.fori_loop(0, ACC // ZB, _za, 0)

  base = c * NWBOX

  def _chunk(t, carry):
    start = base + t * CH
    pltpu.sync_copy(planes.at[0, s, pl.ds(start, CH)], cbuf)
    pltpu.sync_copy(planes.at[1, s, pl.ds(start, CH)], x1b)
    pltpu.sync_copy(planes.at[2, s, pl.ds(start, CH)], y1b)
    pltpu.sync_copy(planes.at[3, s, pl.ds(start, CH)], x2b)
    pltpu.sync_copy(planes.at[4, s, pl.ds(start, CH)], y2b)

    def _sub(j, carry2):
      o = j * SUB
      for g in range(SUB // 16):
        og = o + g * 16
        cf = cbuf[pl.ds(og, 16)]
        x1 = x1b[pl.ds(og, 16)]
        y1 = y1b[pl.ds(og, 16)]
        x2 = x2b[pl.ds(og, 16)]
        y2 = y2b[pl.ds(og, 16)]
        feat_f = jnp.float32(FEAT)
        xi1 = jnp.clip((x1 * feat_f).astype(jnp.int32), 0, FEAT)
        yi1 = jnp.clip((y1 * feat_f).astype(jnp.int32), 0, FEAT)
        xi2 = jnp.clip((x2 * feat_f).astype(jnp.int32), 0, FEAT)
        yi2 = jnp.clip((y2 * feat_f).astype(jnp.int32), 0, FEAT)
        xi2 = jnp.maximum(xi2, xi1)
        yi2 = jnp.maximum(yi2, yi1)
        r1 = soff + yi1 * W
        r2 = soff + yi2 * W
        off = g * 64
        ibuf[pl.ds(off, 16)] = r1 + xi1
        ibuf[pl.ds(off + 16, 16)] = r1 + xi2
        ibuf[pl.ds(off + 32, 16)] = r2 + xi1
        ibuf[pl.ds(off + 48, 16)] = r2 + xi2
        ncf = -cf
        vbuf[pl.ds(off, 16)] = cf
        vbuf[pl.ds(off + 16, 16)] = ncf
        vbuf[pl.ds(off + 32, 16)] = ncf
        vbuf[pl.ds(off + 48, 16)] = cf
      pltpu.sync_copy(vbuf, acc.at[ibuf], add=True)
      return carry2

    lax.fori_loop(0, CH // SUB, _sub, 0)
    return carry

  lax.fori_loop(0, NCHUNK, _chunk, 0)

  def _co(k, carry):
    pltpu.sync_copy(acc.at[pl.ds(soff + k * ZB, ZB)], zbuf)
    pltpu.sync_copy(zbuf, out.at[c, s, pl.ds(k * ZB, ZB)])
    return carry

  lax.fori_loop(0, ACC // ZB, _co, 0)


_sc_scatter = functools.partial(
    pl.kernel,
    out_type=jax.ShapeDtypeStruct((NC, B, ACC), jnp.float32),
    mesh=plsc.VectorSubcoreMesh(
        core_axis_name="c", subcore_axis_name="s", num_cores=NC,
        num_subcores=NS),
    scratch_types=[
        pltpu.VMEM_SHARED((B * ACC,), jnp.float32),
        pltpu.VMEM((CH,), jnp.float32),
        pltpu.VMEM((CH,), jnp.float32),
        pltpu.VMEM((CH,), jnp.float32),
        pltpu.VMEM((CH,), jnp.float32),
        pltpu.VMEM((CH,), jnp.float32),
        pltpu.VMEM((4 * SUB,), jnp.int32),
        pltpu.VMEM((4 * SUB,), jnp.float32),
        pltpu.VMEM((ZB,), jnp.float32),
    ],
)(_sc_scatter_body)


def _tc_cumsum_body(p_ref, o_ref):
  d = (p_ref[0, 0] + p_ref[1, 0]).astype(jnp.bfloat16)
  rows_i = lax.broadcasted_iota(jnp.int32, (HROW, HROW), 0)
  cols_i = lax.broadcasted_iota(jnp.int32, (HROW, HROW), 1)
  ltri = (rows_i >= cols_i).astype(jnp.bfloat16)
  c1 = jnp.dot(ltri, d, preferred_element_type=jnp.float32)
  xs_i = lax.broadcasted_iota(jnp.int32, (W, FEAT), 0)
  js_i = lax.broadcasted_iota(jnp.int32, (W, FEAT), 1)
  utri = (xs_i <= js_i).astype(jnp.bfloat16)
  c2 = jnp.dot(c1.astype(jnp.bfloat16), utri,
               preferred_element_type=jnp.float32)
  v = c2[:FEAT, :]
  o_ref[0] = 1.0 / (1.0 + jnp.exp(-v))


def kernel(preds):
  conf = preds[:, :, 0]
  coords = jnp.moveaxis(preds[:, :, 3:7], -1, 0)
  planes = jnp.concatenate([conf[None], coords], axis=0)
  planes = jnp.pad(planes, ((0, 0), (0, 0), (0, NPAD - N)))
  parts = _sc_scatter(planes)
  parts = parts.reshape(NC, B, HROW, W)
  return pl.pallas_call(
      _tc_cumsum_body,
      grid=(B,),
      in_specs=[
          pl.BlockSpec((NC, 1, HROW, W), lambda b: (0, b, 0, 0)),
      ],
      out_specs=pl.BlockSpec((1, FEAT, FEAT), lambda b: (b, 0, 0)),
      out_shape=jax.ShapeDtypeStruct((B, FEAT, FEAT), jnp.float32),
  )(parts)


# speedup vs baseline: 20.4924x; 20.4924x over previous
import functools

import jax
import jax.numpy as jnp
from jax import lax
from jax.experimental import pallas as pl
from jax.experimental.pallas import tpu as pltpu
from jax.experimental.pallas import tpu_sc as plsc

B = 16
N = 20000
FEAT = 224
W = 256
HROW = 225
ACC = HROW * W
NC = 2
NS = 16
NPAD = 20480
NWBOX = NPAD // NC
CH = 512
NCHUNK = NWBOX // CH
SUB = 32
ZB = 7200


def _sc_scatter_body(planes, out, acc, cbuf, x1b, y1b, x2b, y2b, ibuf, vbuf,
                     zbuf):
  c = lax.axis_index("c")
  s = lax.axis_index("s")
  soff = s * ACC

  def _zb(i, carry):
    zbuf[pl.ds(i * 16, 16)] = jnp.zeros((16,), jnp.float32)
    return carry

  lax.fori_loop(0, ZB // 16, _zb, 0)

  def _za(k, carry):
    pltpu.sync_copy(zbuf, acc.at[pl.ds(soff + k * ZB, ZB)])
    return carry

  lax.fori_loop(0, ACC // ZB, _za, 0)

  base = s * NPAD + c * NWBOX

  def _chunk(t, carry):
    start = base + t * CH
    pltpu.sync_copy(planes.at[pl.ds(0 * B * NPAD + start, CH)], cbuf)
    pltpu.sync_copy(planes.at[pl.ds(1 * B * NPAD + start, CH)], x1b)
    pltpu.sync_copy(planes.at[pl.ds(2 * B * NPAD + start, CH)], y1b)
    pltpu.sync_copy(planes.at[pl.ds(3 * B * NPAD + start, CH)], x2b)
    pltpu.sync_copy(planes.at[pl.ds(4 * B * NPAD + start, CH)], y2b)

    def _sub(j, carry2):
      o = j * SUB
      for g in range(SUB // 16):
        og = o + g * 16
        cf = cbuf[pl.ds(og, 16)]
        x1 = x1b[pl.ds(og, 16)]
        y1 = y1b[pl.ds(og, 16)]
        x2 = x2b[pl.ds(og, 16)]
        y2 = y2b[pl.ds(og, 16)]
        feat_f = jnp.float32(FEAT)
        xi1 = jnp.clip((x1 * feat_f).astype(jnp.int32), 0, FEAT)
        yi1 = jnp.clip((y1 * feat_f).astype(jnp.int32), 0, FEAT)
        xi2 = jnp.clip((x2 * feat_f).astype(jnp.int32), 0, FEAT)
        yi2 = jnp.clip((y2 * feat_f).astype(jnp.int32), 0, FEAT)
        xi2 = jnp.maximum(xi2, xi1)
        yi2 = jnp.maximum(yi2, yi1)
        r1 = soff + yi1 * W
        r2 = soff + yi2 * W
        off = g * 64
        ibuf[pl.ds(off, 16)] = r1 + xi1
        ibuf[pl.ds(off + 16, 16)] = r1 + xi2
        ibuf[pl.ds(off + 32, 16)] = r2 + xi1
        ibuf[pl.ds(off + 48, 16)] = r2 + xi2
        ncf = -cf
        vbuf[pl.ds(off, 16)] = cf
        vbuf[pl.ds(off + 16, 16)] = ncf
        vbuf[pl.ds(off + 32, 16)] = ncf
        vbuf[pl.ds(off + 48, 16)] = cf
      pltpu.sync_copy(vbuf, acc.at[ibuf], add=True)
      return carry2

    lax.fori_loop(0, CH // SUB, _sub, 0)
    return carry

  lax.fori_loop(0, NCHUNK, _chunk, 0)

  obase = (c * B + s) * ACC

  def _co(k, carry):
    pltpu.sync_copy(acc.at[pl.ds(soff + k * ZB, ZB)], zbuf)
    pltpu.sync_copy(zbuf, out.at[pl.ds(obase + k * ZB, ZB)])
    return carry

  lax.fori_loop(0, ACC // ZB, _co, 0)


_sc_scatter = functools.partial(
    pl.kernel,
    out_type=jax.ShapeDtypeStruct((NC * B * ACC,), jnp.float32),
    mesh=plsc.VectorSubcoreMesh(
        core_axis_name="c", subcore_axis_name="s", num_cores=NC,
        num_subcores=NS),
    scratch_types=[
        pltpu.VMEM_SHARED((B * ACC,), jnp.float32),
        pltpu.VMEM((CH,), jnp.float32),
        pltpu.VMEM((CH,), jnp.float32),
        pltpu.VMEM((CH,), jnp.float32),
        pltpu.VMEM((CH,), jnp.float32),
        pltpu.VMEM((CH,), jnp.float32),
        pltpu.VMEM((4 * SUB,), jnp.int32),
        pltpu.VMEM((4 * SUB,), jnp.float32),
        pltpu.VMEM((ZB,), jnp.float32),
    ],
)(_sc_scatter_body)


def _tc_cumsum_body(p_ref, o_ref):
  d = (p_ref[0, 0] + p_ref[1, 0]).astype(jnp.bfloat16)
  rows_i = lax.broadcasted_iota(jnp.int32, (HROW, HROW), 0)
  cols_i = lax.broadcasted_iota(jnp.int32, (HROW, HROW), 1)
  ltri = (rows_i >= cols_i).astype(jnp.bfloat16)
  c1 = jnp.dot(ltri, d, preferred_element_type=jnp.float32)
  xs_i = lax.broadcasted_iota(jnp.int32, (W, FEAT), 0)
  js_i = lax.broadcasted_iota(jnp.int32, (W, FEAT), 1)
  utri = (xs_i <= js_i).astype(jnp.bfloat16)
  c2 = jnp.dot(c1.astype(jnp.bfloat16), utri,
               preferred_element_type=jnp.float32)
  v = c2[:FEAT, :]
  o_ref[0] = 1.0 / (1.0 + jnp.exp(-v))


def kernel(preds):
  conf = preds[:, :, 0]
  coords = jnp.moveaxis(preds[:, :, 3:7], -1, 0)
  planes = jnp.concatenate([conf[None], coords], axis=0)
  planes = jnp.pad(planes, ((0, 0), (0, 0), (0, NPAD - N)))
  parts = _sc_scatter(planes.reshape(-1))
  parts = parts.reshape(NC, B, HROW, W)
  return pl.pallas_call(
      _tc_cumsum_body,
      grid=(B,),
      in_specs=[
          pl.BlockSpec((NC, 1, HROW, W), lambda b: (0, b, 0, 0)),
      ],
      out_specs=pl.BlockSpec((1, FEAT, FEAT), lambda b: (b, 0, 0)),
      out_shape=jax.ShapeDtypeStruct((B, FEAT, FEAT), jnp.float32),
  )(parts)


# speedup vs baseline: 31.4666x; 1.5355x over previous
import functools

import jax
import jax.numpy as jnp
from jax import lax
from jax.experimental import pallas as pl
from jax.experimental.pallas import tpu as pltpu
from jax.experimental.pallas import tpu_sc as plsc

B = 16
N = 20000
FEAT = 224
W = 256
HROW = 225
ACC = HROW * W
NC = 2
NS = 16
NPAD = 20480
NWBOX = NPAD // NC
CH = 512
NCHUNK = NWBOX // CH
SUB = 32
NSUB = CH // SUB
ZB = 7200


def _sc_scatter_body(planes, out, acc, cb, x1b, y1b, x2b, y2b, ibuf, vbuf,
                     zbuf, sem_in, sem_sc):
  c = lax.axis_index("c")
  s = lax.axis_index("s")
  soff = s * ACC

  def _zb(i, carry):
    zbuf[pl.ds(i * 16, 16)] = jnp.zeros((16,), jnp.float32)
    return carry

  lax.fori_loop(0, ZB // 16, _zb, 0)

  def _za(k, carry):
    pltpu.sync_copy(zbuf, acc.at[pl.ds(soff + k * ZB, ZB)])
    return carry

  lax.fori_loop(0, ACC // ZB, _za, 0)

  base = s * NPAD + c * NWBOX
  feat_f = jnp.float32(FEAT)

  def _chunk(t, carry):
    start = base + t * CH
    stage = [
        pltpu.async_copy(planes.at[pl.ds(p * B * NPAD + start, CH)], dst,
                         sem_in)
        for p, dst in enumerate((cb, x1b, y1b, x2b, y2b))
    ]
    for d in stage:
      d.wait()
    for j in range(NSUB):
      for g in range(SUB // 16):
        og = j * SUB + g * 16
        cf = cb[pl.ds(og, 16)]
        x1 = x1b[pl.ds(og, 16)]
        y1 = y1b[pl.ds(og, 16)]
        x2 = x2b[pl.ds(og, 16)]
        y2 = y2b[pl.ds(og, 16)]
        xi1 = jnp.clip((x1 * feat_f).astype(jnp.int32), 0, FEAT)
        yi1 = jnp.clip((y1 * feat_f).astype(jnp.int32), 0, FEAT)
        xi2 = jnp.clip((x2 * feat_f).astype(jnp.int32), 0, FEAT)
        yi2 = jnp.clip((y2 * feat_f).astype(jnp.int32), 0, FEAT)
        xi2 = jnp.maximum(xi2, xi1)
        yi2 = jnp.maximum(yi2, yi1)
        r1 = soff + yi1 * W
        r2 = soff + yi2 * W
        off = g * 64
        ibuf[j, pl.ds(off, 16)] = r1 + xi1
        ibuf[j, pl.ds(off + 16, 16)] = r1 + xi2
        ibuf[j, pl.ds(off + 32, 16)] = r2 + xi1
        ibuf[j, pl.ds(off + 48, 16)] = r2 + xi2
        ncf = -cf
        vbuf[j, pl.ds(off, 16)] = cf
        vbuf[j, pl.ds(off + 16, 16)] = ncf
        vbuf[j, pl.ds(off + 32, 16)] = ncf
        vbuf[j, pl.ds(off + 48, 16)] = cf
    scat = [
        pltpu.async_copy(vbuf.at[j], acc.at[ibuf.at[j]], sem_sc, add=True)
        for j in range(NSUB)
    ]
    for d in scat:
      d.wait()
    return carry

  lax.fori_loop(0, NCHUNK, _chunk, 0)

  obase = (c * B + s) * ACC

  def _co(k, carry):
    pltpu.sync_copy(acc.at[pl.ds(soff + k * ZB, ZB)], zbuf)
    pltpu.sync_copy(zbuf, out.at[pl.ds(obase + k * ZB, ZB)])
    return carry

  lax.fori_loop(0, ACC // ZB, _co, 0)


_sc_scatter = functools.partial(
    pl.kernel,
    out_type=jax.ShapeDtypeStruct((NC * B * ACC,), jnp.float32),
    mesh=plsc.VectorSubcoreMesh(
        core_axis_name="c", subcore_axis_name="s", num_cores=NC,
        num_subcores=NS),
    scratch_types=[
        pltpu.VMEM_SHARED((B * ACC,), jnp.float32),
        pltpu.VMEM((CH,), jnp.float32),
        pltpu.VMEM((CH,), jnp.float32),
        pltpu.VMEM((CH,), jnp.float32),
        pltpu.VMEM((CH,), jnp.float32),
        pltpu.VMEM((CH,), jnp.float32),
        pltpu.VMEM((NSUB, 4 * SUB), jnp.int32),
        pltpu.VMEM((NSUB, 4 * SUB), jnp.float32),
        pltpu.VMEM((ZB,), jnp.float32),
        pltpu.SemaphoreType.DMA,
        pltpu.SemaphoreType.DMA,
    ],
)(_sc_scatter_body)


def _tc_cumsum_body(p_ref, o_ref):
  d = (p_ref[0, 0] + p_ref[1, 0]).astype(jnp.bfloat16)
  rows_i = lax.broadcasted_iota(jnp.int32, (HROW, HROW), 0)
  cols_i = lax.broadcasted_iota(jnp.int32, (HROW, HROW), 1)
  ltri = (rows_i >= cols_i).astype(jnp.bfloat16)
  c1 = jnp.dot(ltri, d, preferred_element_type=jnp.float32)
  xs_i = lax.broadcasted_iota(jnp.int32, (W, FEAT), 0)
  js_i = lax.broadcasted_iota(jnp.int32, (W, FEAT), 1)
  utri = (xs_i <= js_i).astype(jnp.bfloat16)
  c2 = jnp.dot(c1.astype(jnp.bfloat16), utri,
               preferred_element_type=jnp.float32)
  v = c2[:FEAT, :]
  o_ref[0] = 1.0 / (1.0 + jnp.exp(-v))


def kernel(preds):
  conf = preds[:, :, 0]
  coords = jnp.moveaxis(preds[:, :, 3:7], -1, 0)
  planes = jnp.concatenate([conf[None], coords], axis=0)
  planes = jnp.pad(planes, ((0, 0), (0, 0), (0, NPAD - N)))
  parts = _sc_scatter(planes.reshape(-1))
  parts = parts.reshape(NC, B, HROW, W)
  return pl.pallas_call(
      _tc_cumsum_body,
      grid=(B,),
      in_specs=[
          pl.BlockSpec((NC, 1, HROW, W), lambda b: (0, b, 0, 0)),
      ],
      out_specs=pl.BlockSpec((1, FEAT, FEAT), lambda b: (b, 0, 0)),
      out_shape=jax.ShapeDtypeStruct((B, FEAT, FEAT), jnp.float32),
  )(parts)
